# Initial kernel scaffold; baseline (speedup 1.0000x reference)
import functools

import jax
import jax.numpy as jnp
from jax import lax
from jax.experimental import pallas as pl
from jax.experimental.pallas import tpu as pltpu
from jax.experimental.pallas import tpu_sc as plsc

N = 10000
E = 320000
D_IN = 128
H = 32
Z = 16

NC = 2
NS = 16
NW = NC * NS
EPC = E // NC
EPT = E // NW
B = 80
NCH = EPT // B
DW = 8

_mesh = plsc.VectorSubcoreMesh(core_axis_name="c", subcore_axis_name="s")


@functools.partial(
    pl.kernel,
    out_type=jax.ShapeDtypeStruct((NC, N, DW), jnp.float32),
    mesh=_mesh,
    scratch_types=[
        pltpu.VMEM((B, DW), jnp.float32),
        pltpu.VMEM((B,), jnp.int32),
        pltpu.VMEM_SHARED((N, DW), jnp.float32),
    ],
)
def _sc_deg(ei_hbm, ones_hbm, zeros_hbm, degp_hbm, onesb, idxb, degsh):
    c = lax.axis_index("c")
    s = lax.axis_index("s")
    @pl.when(s < 10)
    def _():
        pltpu.sync_copy(zeros_hbm.at[pl.ds(s * 1000, 1000)],
                        degsh.at[pl.ds(s * 1000, 1000)])
    pltpu.sync_copy(ones_hbm, onesb)
    plsc.subcore_barrier()

    edge_base = c * EPC + s * EPT

    def body(i, carry):
        pltpu.sync_copy(ei_hbm.at[1, pl.ds(edge_base + i * B, B)], idxb)
        pltpu.sync_copy(onesb, degsh.at[idxb], add=True)
        return carry

    lax.fori_loop(0, NCH, body, 0)
    plsc.subcore_barrier()

    @pl.when(s < 10)
    def _():
        pltpu.sync_copy(degsh.at[pl.ds(s * 1000, 1000)],
                        degp_hbm.at[c, pl.ds(s * 1000, 1000)])


@functools.partial(
    pl.kernel,
    out_type=jax.ShapeDtypeStruct((NC, N, H), jnp.float32),
    mesh=_mesh,
    scratch_types=[
        pltpu.VMEM((B,), jnp.int32),
        pltpu.VMEM((B,), jnp.int32),
        pltpu.VMEM((B, H), jnp.float32),
        pltpu.VMEM_SHARED((N, H), jnp.float32),
    ],
)
def _sc_prop(y_hbm, ei_hbm, zeros_hbm, p_hbm, srcb, dstb, rows, accsh):
    c = lax.axis_index("c")
    s = lax.axis_index("s")
    row_base = s * (N // NS)
    pltpu.sync_copy(zeros_hbm.at[pl.ds(row_base, N // NS)],
                    accsh.at[pl.ds(row_base, N // NS)])
    plsc.subcore_barrier()

    edge_base = c * EPC + s * EPT

    def body(i, carry):
        off = edge_base + i * B
        pltpu.sync_copy(ei_hbm.at[0, pl.ds(off, B)], srcb)
        pltpu.sync_copy(ei_hbm.at[1, pl.ds(off, B)], dstb)
        pltpu.sync_copy(y_hbm.at[srcb], rows)
        pltpu.sync_copy(rows, accsh.at[dstb], add=True)
        return carry

    lax.fori_loop(0, NCH, body, 0)
    plsc.subcore_barrier()

    pltpu.sync_copy(accsh.at[pl.ds(row_base, N // NS)],
                    p_hbm.at[c, pl.ds(row_base, N // NS)])


def _tc_pre_body(x_ref, w1_ref, degp_ref, y1_ref, dis_ref):
    deg = degp_ref[0, :, 0:1] + degp_ref[1, :, 0:1] + 1.0
    dis = lax.rsqrt(deg)
    xw = jnp.dot(x_ref[...], w1_ref[...], preferred_element_type=jnp.float32)
    y1_ref[...] = dis * xw
    dis_ref[...] = dis


def _tc_mid_body(p_ref, y1_ref, dis_ref, y2_ref):
    dis = dis_ref[...]
    t = p_ref[0] + p_ref[1] + y1_ref[...]
    y2_ref[...] = dis * jnp.maximum(dis * t, 0.0)


def _tc_post_body(p_ref, y2_ref, dis_ref, wm_ref, wv_ref, zm_ref, zv_ref):
    sfin = dis_ref[...] * (p_ref[0] + p_ref[1] + y2_ref[...])
    zm_ref[...] = jnp.dot(sfin, wm_ref[...], preferred_element_type=jnp.float32)
    zv_ref[...] = jnp.dot(sfin, wv_ref[...], preferred_element_type=jnp.float32)


_tc_pre = pl.pallas_call(
    _tc_pre_body,
    out_shape=(
        jax.ShapeDtypeStruct((N, H), jnp.float32),
        jax.ShapeDtypeStruct((N, 1), jnp.float32),
    ),
)

_tc_mid = pl.pallas_call(
    _tc_mid_body,
    out_shape=jax.ShapeDtypeStruct((N, H), jnp.float32),
)

_tc_post = pl.pallas_call(
    _tc_post_body,
    out_shape=(
        jax.ShapeDtypeStruct((N, Z), jnp.float32),
        jax.ShapeDtypeStruct((N, Z), jnp.float32),
    ),
)


@jax.jit
def kernel(features, edge_index, W1, W_mean, W_var):
    ones_rows = jnp.ones((B, DW), jnp.float32)
    zeros_deg = jnp.zeros((N, DW), jnp.float32)
    zeros_acc = jnp.zeros((N, H), jnp.float32)

    degp = _sc_deg(edge_index, ones_rows, zeros_deg)
    y1, dis = _tc_pre(features, W1, degp)
    p1 = _sc_prop(y1, edge_index, zeros_acc)
    y2 = _tc_mid(p1, y1, dis)
    p2 = _sc_prop(y2, edge_index, zeros_acc)
    z_mean, z_var = _tc_post(p2, y2, dis, W_mean, W_var)
    return (z_mean, z_var)


# speedup vs baseline: 19.1541x; 19.1541x over previous
import functools

import jax
import jax.numpy as jnp
from jax import lax
from jax.experimental import pallas as pl
from jax.experimental.pallas import tpu as pltpu
from jax.experimental.pallas import tpu_sc as plsc

N = 10000
E = 320000
D_IN = 128
H = 32
Z = 16

NC = 2
NS = 16
NW = NC * NS
EPC = E // NC
EPT = E // NW
B = 80
NCH = EPT // B
DW = 8

_mesh = plsc.VectorSubcoreMesh(core_axis_name="c", subcore_axis_name="s")


@functools.partial(
    pl.kernel,
    out_type=jax.ShapeDtypeStruct((NC, N, DW), jnp.float32),
    mesh=_mesh,
    compiler_params=pltpu.CompilerParams(use_tc_tiling_on_sc=False),
    scratch_types=[
        pltpu.VMEM((B, DW), jnp.float32),
        pltpu.VMEM((B,), jnp.int32),
        pltpu.VMEM_SHARED((N, DW), jnp.float32),
    ],
)
def _sc_deg(dst_hbm, ones_hbm, zeros_hbm, degp_hbm, onesb, idxb, degsh):
    c = lax.axis_index("c")
    s = lax.axis_index("s")
    @pl.when(s < 10)
    def _():
        pltpu.sync_copy(zeros_hbm.at[pl.ds(s * 1000, 1000)],
                        degsh.at[pl.ds(s * 1000, 1000)])
    pltpu.sync_copy(ones_hbm, onesb)
    plsc.subcore_barrier()

    edge_base = c * EPC + s * EPT

    def body(i, carry):
        pltpu.sync_copy(dst_hbm.at[pl.ds(edge_base + i * B, B)], idxb)
        pltpu.sync_copy(onesb, degsh.at[idxb], add=True)
        return carry

    lax.fori_loop(0, NCH, body, 0)
    plsc.subcore_barrier()

    @pl.when(s < 10)
    def _():
        pltpu.sync_copy(degsh.at[pl.ds(s * 1000, 1000)],
                        degp_hbm.at[c, pl.ds(s * 1000, 1000)])


@functools.partial(
    pl.kernel,
    out_type=jax.ShapeDtypeStruct((NC, N, H), jnp.float32),
    mesh=_mesh,
    compiler_params=pltpu.CompilerParams(use_tc_tiling_on_sc=False),
    scratch_types=[
        pltpu.VMEM((B,), jnp.int32),
        pltpu.VMEM((B,), jnp.int32),
        pltpu.VMEM((B, H), jnp.float32),
        pltpu.VMEM_SHARED((N, H), jnp.float32),
    ],
)
def _sc_prop(y_hbm, src_hbm, dst_hbm, zeros_hbm, p_hbm, srcb, dstb, rows, accsh):
    c = lax.axis_index("c")
    s = lax.axis_index("s")
    row_base = s * 1000

    @pl.when(s < 10)
    def _():
        pltpu.sync_copy(zeros_hbm.at[pl.ds(row_base, 1000)],
                        accsh.at[pl.ds(row_base, 1000)])
    plsc.subcore_barrier()

    edge_base = c * EPC + s * EPT

    def body(i, carry):
        off = edge_base + i * B
        pltpu.sync_copy(src_hbm.at[pl.ds(off, B)], srcb)
        pltpu.sync_copy(dst_hbm.at[pl.ds(off, B)], dstb)
        pltpu.sync_copy(y_hbm.at[srcb], rows)
        pltpu.sync_copy(rows, accsh.at[dstb], add=True)
        return carry

    lax.fori_loop(0, NCH, body, 0)
    plsc.subcore_barrier()

    @pl.when(s < 10)
    def _():
        pltpu.sync_copy(accsh.at[pl.ds(row_base, 1000)],
                        p_hbm.at[c, pl.ds(row_base, 1000)])


def _tc_pre_body(x_ref, w1_ref, degp_ref, y1_ref, dis_ref):
    deg = degp_ref[0, :, 0:1] + degp_ref[1, :, 0:1] + 1.0
    dis = lax.rsqrt(deg)
    xw = jnp.dot(x_ref[...], w1_ref[...], preferred_element_type=jnp.float32)
    y1_ref[...] = dis * xw
    dis_ref[...] = dis


def _tc_mid_body(p_ref, y1_ref, dis_ref, y2_ref):
    dis = dis_ref[...]
    t = p_ref[0] + p_ref[1] + y1_ref[...]
    y2_ref[...] = dis * jnp.maximum(dis * t, 0.0)


def _tc_post_body(p_ref, y2_ref, dis_ref, wm_ref, wv_ref, zm_ref, zv_ref):
    sfin = dis_ref[...] * (p_ref[0] + p_ref[1] + y2_ref[...])
    zm_ref[...] = jnp.dot(sfin, wm_ref[...], preferred_element_type=jnp.float32)
    zv_ref[...] = jnp.dot(sfin, wv_ref[...], preferred_element_type=jnp.float32)


_tc_pre = pl.pallas_call(
    _tc_pre_body,
    out_shape=(
        jax.ShapeDtypeStruct((N, H), jnp.float32),
        jax.ShapeDtypeStruct((N, 1), jnp.float32),
    ),
)

_tc_mid = pl.pallas_call(
    _tc_mid_body,
    out_shape=jax.ShapeDtypeStruct((N, H), jnp.float32),
)

_tc_post = pl.pallas_call(
    _tc_post_body,
    out_shape=(
        jax.ShapeDtypeStruct((N, Z), jnp.float32),
        jax.ShapeDtypeStruct((N, Z), jnp.float32),
    ),
)


@jax.jit
def kernel(features, edge_index, W1, W_mean, W_var):
    ones_rows = jnp.ones((B, DW), jnp.float32)
    zeros_deg = jnp.zeros((N, DW), jnp.float32)
    zeros_acc = jnp.zeros((N, H), jnp.float32)

    src = edge_index[0]
    dst = edge_index[1]
    degp = _sc_deg(dst, ones_rows, zeros_deg)
    y1, dis = _tc_pre(features, W1, degp)
    p1 = _sc_prop(y1, src, dst, zeros_acc)
    y2 = _tc_mid(p1, y1, dis)
    p2 = _sc_prop(y2, src, dst, zeros_acc)
    z_mean, z_var = _tc_post(p2, y2, dis, W_mean, W_var)
    return (z_mean, z_var)


# speedup vs baseline: 62.9828x; 3.2882x over previous
import functools

import jax
import jax.numpy as jnp
from jax import lax
from jax.experimental import pallas as pl
from jax.experimental.pallas import tpu as pltpu
from jax.experimental.pallas import tpu_sc as plsc

N = 10000
E = 320000
D_IN = 128
H = 32
Z = 16

NC = 2
NS = 16
NW = NC * NS
B = 125
NCHT = E // B // NW
DW = 8
K = 4
A = 2
KD = 8

_mesh = plsc.VectorSubcoreMesh(core_axis_name="c", subcore_axis_name="s")


@functools.partial(
    pl.kernel,
    out_type=jax.ShapeDtypeStruct((NC, N, DW), jnp.float32),
    mesh=_mesh,
    compiler_params=pltpu.CompilerParams(use_tc_tiling_on_sc=False),
    scratch_types=[
        pltpu.VMEM((B, DW), jnp.float32),
        pltpu.VMEM((NCHT, B), jnp.int32),
        pltpu.VMEM_SHARED((N, DW), jnp.float32),
        pltpu.SemaphoreType.DMA((KD,)),
    ],
)
def _sc_deg(dst2_hbm, ones_hbm, zeros_hbm, degp_hbm, onesb, dsti, degsh, ssem):
    c = lax.axis_index("c")
    s = lax.axis_index("s")
    w = c * NS + s
    @pl.when(s < 10)
    def _():
        pltpu.sync_copy(zeros_hbm.at[pl.ds(s * 1000, 1000)],
                        degsh.at[pl.ds(s * 1000, 1000)])
    pltpu.sync_copy(ones_hbm, onesb)
    pltpu.sync_copy(dst2_hbm.at[pl.ds(w * NCHT, NCHT)], dsti)
    plsc.subcore_barrier()

    @pl.loop(0, NCHT, step=KD)
    def _(j0):
        for b in range(KD):
            j = j0 + b

            @pl.when(j >= KD)
            def _():
                pltpu.make_async_copy(
                    onesb, degsh.at[dsti.at[j - KD]], ssem.at[b]).wait()

            pltpu.async_copy(onesb, degsh.at[dsti.at[j]], ssem.at[b],
                             add=True)

    for b in range(KD):
        pltpu.make_async_copy(
            onesb, degsh.at[dsti.at[NCHT - KD + b]], ssem.at[b]).wait()

    plsc.subcore_barrier()

    @pl.when(s < 10)
    def _():
        pltpu.sync_copy(degsh.at[pl.ds(s * 1000, 1000)],
                        degp_hbm.at[c, pl.ds(s * 1000, 1000)])


@functools.partial(
    pl.kernel,
    out_type=jax.ShapeDtypeStruct((NC, N, H), jnp.float32),
    mesh=_mesh,
    compiler_params=pltpu.CompilerParams(use_tc_tiling_on_sc=False),
    scratch_types=[
        pltpu.VMEM((NCHT, B), jnp.int32),
        pltpu.VMEM((NCHT, B), jnp.int32),
        pltpu.VMEM((K, B, H), jnp.float32),
        pltpu.VMEM_SHARED((N, H), jnp.float32),
        pltpu.SemaphoreType.DMA((K,)),
        pltpu.SemaphoreType.DMA((K,)),
    ],
)
def _sc_prop(y_hbm, src2_hbm, dst2_hbm, zeros_hbm, p_hbm,
             srci, dsti, rows, accsh, gsem, ssem):
    c = lax.axis_index("c")
    s = lax.axis_index("s")
    w = c * NS + s
    row_base = s * 1000

    @pl.when(s < 10)
    def _():
        pltpu.sync_copy(zeros_hbm.at[pl.ds(row_base, 1000)],
                        accsh.at[pl.ds(row_base, 1000)])
    pltpu.sync_copy(src2_hbm.at[pl.ds(w * NCHT, NCHT)], srci)
    pltpu.sync_copy(dst2_hbm.at[pl.ds(w * NCHT, NCHT)], dsti)
    plsc.subcore_barrier()

    for j in range(A):
        pltpu.async_copy(y_hbm.at[srci.at[j]], rows.at[j % K], gsem.at[j % K])

    @pl.loop(0, NCHT, step=K)
    def _(j0):
        for b in range(K):
            j = j0 + b
            b2 = (b + A) % K

            @pl.when(j + A >= K)
            def _():
                pltpu.make_async_copy(
                    rows.at[b2], accsh.at[dsti.at[j + A - K]],
                    ssem.at[b2]).wait()

            @pl.when(j + A < NCHT)
            def _():
                pltpu.async_copy(y_hbm.at[srci.at[j + A]], rows.at[b2],
                                 gsem.at[b2])

            pltpu.make_async_copy(
                y_hbm.at[srci.at[j]], rows.at[b], gsem.at[b]).wait()
            pltpu.async_copy(rows.at[b], accsh.at[dsti.at[j]], ssem.at[b],
                             add=True)

    for i in range(K - A):
        j = NCHT - (K - A) + i
        b = j % K
        pltpu.make_async_copy(
            rows.at[b], accsh.at[dsti.at[j]], ssem.at[b]).wait()

    plsc.subcore_barrier()

    @pl.when(s < 10)
    def _():
        pltpu.sync_copy(accsh.at[pl.ds(row_base, 1000)],
                        p_hbm.at[c, pl.ds(row_base, 1000)])


def _tc_pre_body(x_ref, w1_ref, degp_ref, y1_ref, dis_ref):
    deg = degp_ref[0, :, 0:1] + degp_ref[1, :, 0:1] + 1.0
    dis = lax.rsqrt(deg)
    xw = jnp.dot(x_ref[...], w1_ref[...], preferred_element_type=jnp.float32)
    y1_ref[...] = dis * xw
    dis_ref[...] = dis


def _tc_mid_body(p_ref, y1_ref, dis_ref, y2_ref):
    dis = dis_ref[...]
    t = p_ref[0] + p_ref[1] + y1_ref[...]
    y2_ref[...] = dis * jnp.maximum(dis * t, 0.0)


def _tc_post_body(p_ref, y2_ref, dis_ref, wm_ref, wv_ref, zm_ref, zv_ref):
    sfin = dis_ref[...] * (p_ref[0] + p_ref[1] + y2_ref[...])
    zm_ref[...] = jnp.dot(sfin, wm_ref[...], preferred_element_type=jnp.float32)
    zv_ref[...] = jnp.dot(sfin, wv_ref[...], preferred_element_type=jnp.float32)


_tc_pre = pl.pallas_call(
    _tc_pre_body,
    out_shape=(
        jax.ShapeDtypeStruct((N, H), jnp.float32),
        jax.ShapeDtypeStruct((N, 1), jnp.float32),
    ),
)

_tc_mid = pl.pallas_call(
    _tc_mid_body,
    out_shape=jax.ShapeDtypeStruct((N, H), jnp.float32),
)

_tc_post = pl.pallas_call(
    _tc_post_body,
    out_shape=(
        jax.ShapeDtypeStruct((N, Z), jnp.float32),
        jax.ShapeDtypeStruct((N, Z), jnp.float32),
    ),
)


@jax.jit
def kernel(features, edge_index, W1, W_mean, W_var):
    ones_rows = jnp.ones((B, DW), jnp.float32)
    zeros_deg = jnp.zeros((N, DW), jnp.float32)
    zeros_acc = jnp.zeros((N, H), jnp.float32)

    src2 = edge_index[0].reshape(E // B, B)
    dst2 = edge_index[1].reshape(E // B, B)
    degp = _sc_deg(dst2, ones_rows, zeros_deg)
    y1, dis = _tc_pre(features, W1, degp)
    p1 = _sc_prop(y1, src2, dst2, zeros_acc)
    y2 = _tc_mid(p1, y1, dis)
    p2 = _sc_prop(y2, src2, dst2, zeros_acc)
    z_mean, z_var = _tc_post(p2, y2, dis, W_mean, W_var)
    return (z_mean, z_var)


# speedup vs baseline: 74.9163x; 1.1895x over previous
import functools

import jax
import jax.numpy as jnp
from jax import lax
from jax.experimental import pallas as pl
from jax.experimental.pallas import tpu as pltpu
from jax.experimental.pallas import tpu_sc as plsc

N = 10000
E = 320000
D_IN = 128
H = 32
Z = 16

NC = 2
NS = 16
NW = NC * NS
B = 125
NCHT = E // B // NW
DW = 8
K = 4
A = 2
KD = 8
FR = N * H // 128

_mesh = plsc.VectorSubcoreMesh(core_axis_name="c", subcore_axis_name="s")


@functools.partial(
    pl.kernel,
    out_type=jax.ShapeDtypeStruct((NC, N, DW), jnp.float32),
    mesh=_mesh,
    compiler_params=pltpu.CompilerParams(use_tc_tiling_on_sc=False),
    scratch_types=[
        pltpu.VMEM((B, DW), jnp.float32),
        pltpu.VMEM((NCHT, B), jnp.int32),
        pltpu.VMEM_SHARED((N, DW), jnp.float32),
        pltpu.SemaphoreType.DMA((KD,)),
    ],
)
def _sc_deg(ei_hbm, ones_hbm, zeros_hbm, degp_hbm, onesb, dsti, degsh, ssem):
    c = lax.axis_index("c")
    s = lax.axis_index("s")
    w = c * NS + s
    @pl.when(s < 10)
    def _():
        pltpu.sync_copy(zeros_hbm.at[pl.ds(s * 1000, 1000)],
                        degsh.at[pl.ds(s * 1000, 1000)])
    pltpu.sync_copy(ones_hbm, onesb)
    pltpu.sync_copy(ei_hbm.at[1, pl.ds(w * NCHT, NCHT)], dsti)
    plsc.subcore_barrier()

    @pl.loop(0, NCHT, step=KD)
    def _(j0):
        for b in range(KD):
            j = j0 + b

            @pl.when(j >= KD)
            def _():
                pltpu.make_async_copy(
                    onesb, degsh.at[dsti.at[j - KD]], ssem.at[b]).wait()

            pltpu.async_copy(onesb, degsh.at[dsti.at[j]], ssem.at[b],
                             add=True)

    for b in range(KD):
        pltpu.make_async_copy(
            onesb, degsh.at[dsti.at[NCHT - KD + b]], ssem.at[b]).wait()

    plsc.subcore_barrier()

    @pl.when(s < 10)
    def _():
        pltpu.sync_copy(degsh.at[pl.ds(s * 1000, 1000)],
                        degp_hbm.at[c, pl.ds(s * 1000, 1000)])


@functools.partial(
    pl.kernel,
    out_type=jax.ShapeDtypeStruct((NC, N, H), jnp.float32),
    mesh=_mesh,
    compiler_params=pltpu.CompilerParams(use_tc_tiling_on_sc=False),
    scratch_types=[
        pltpu.VMEM((NCHT, B), jnp.int32),
        pltpu.VMEM((NCHT, B), jnp.int32),
        pltpu.VMEM((K, B, H), jnp.float32),
        pltpu.VMEM_SHARED((N, H), jnp.float32),
        pltpu.SemaphoreType.DMA((K,)),
        pltpu.SemaphoreType.DMA((K,)),
    ],
)
def _sc_prop(y_hbm, ei_hbm, zeros_hbm, p_hbm,
             srci, dsti, rows, accsh, gsem, ssem):
    c = lax.axis_index("c")
    s = lax.axis_index("s")
    w = c * NS + s
    row_base = s * 1000

    @pl.when(jnp.logical_and(s < 10, c == 0))
    def _():
        pltpu.sync_copy(y_hbm.at[pl.ds(row_base, 1000)],
                        accsh.at[pl.ds(row_base, 1000)])

    @pl.when(jnp.logical_and(s < 10, c == 1))
    def _():
        pltpu.sync_copy(zeros_hbm.at[pl.ds(row_base, 1000)],
                        accsh.at[pl.ds(row_base, 1000)])

    pltpu.sync_copy(ei_hbm.at[0, pl.ds(w * NCHT, NCHT)], srci)
    pltpu.sync_copy(ei_hbm.at[1, pl.ds(w * NCHT, NCHT)], dsti)
    plsc.subcore_barrier()

    for j in range(A):
        pltpu.async_copy(y_hbm.at[srci.at[j]], rows.at[j % K], gsem.at[j % K])

    @pl.loop(0, NCHT, step=K)
    def _(j0):
        for b in range(K):
            j = j0 + b
            b2 = (b + A) % K

            @pl.when(j + A >= K)
            def _():
                pltpu.make_async_copy(
                    rows.at[b2], accsh.at[dsti.at[j + A - K]],
                    ssem.at[b2]).wait()

            @pl.when(j + A < NCHT)
            def _():
                pltpu.async_copy(y_hbm.at[srci.at[j + A]], rows.at[b2],
                                 gsem.at[b2])

            pltpu.make_async_copy(
                y_hbm.at[srci.at[j]], rows.at[b], gsem.at[b]).wait()
            pltpu.async_copy(rows.at[b], accsh.at[dsti.at[j]], ssem.at[b],
                             add=True)

    for i in range(K - A):
        j = NCHT - (K - A) + i
        b = j % K
        pltpu.make_async_copy(
            rows.at[b], accsh.at[dsti.at[j]], ssem.at[b]).wait()

    plsc.subcore_barrier()

    @pl.when(s < 10)
    def _():
        pltpu.sync_copy(accsh.at[pl.ds(row_base, 1000)],
                        p_hbm.at[c, pl.ds(row_base, 1000)])


def _tc_pre_body(x_ref, w1_ref, degp_ref, y1_ref, dis_ref):
    deg = degp_ref[0, :, 0:1] + degp_ref[1, :, 0:1] + 1.0
    dis = lax.rsqrt(deg)
    xw = jnp.dot(x_ref[...], w1_ref[...], preferred_element_type=jnp.float32)
    y1_ref[...] = dis * xw
    dis_ref[...] = jnp.broadcast_to(dis, (N, H))


def _tc_mid_body(p_ref, dis_ref, y2_ref):
    dis = dis_ref[...]
    t = p_ref[0] + p_ref[1]
    y2_ref[...] = dis * jnp.maximum(dis * t, 0.0)


def _tc_post_body(p_ref, dis_ref, wbd_ref, zf_ref):
    sfin = dis_ref[...] * (p_ref[0] + p_ref[1])
    zf_ref[...] = jnp.dot(sfin, wbd_ref[...],
                          preferred_element_type=jnp.float32)


_tc_pre = pl.pallas_call(
    _tc_pre_body,
    out_shape=(
        jax.ShapeDtypeStruct((N, H), jnp.float32),
        jax.ShapeDtypeStruct((N, H), jnp.float32),
    ),
)

_tc_mid = pl.pallas_call(
    _tc_mid_body,
    out_shape=jax.ShapeDtypeStruct((FR, 128), jnp.float32),
)

_tc_post = pl.pallas_call(
    _tc_post_body,
    out_shape=jax.ShapeDtypeStruct((FR, 128), jnp.float32),
)


@jax.jit
def kernel(features, edge_index, W1, W_mean, W_var):
    ones_rows = jnp.ones((B, DW), jnp.float32)
    zeros_deg = jnp.zeros((N, DW), jnp.float32)
    zeros_acc = jnp.zeros((N, H), jnp.float32)
    wcat = jnp.concatenate([W_mean, W_var], axis=1)
    wbd = jnp.kron(jnp.eye(4, dtype=jnp.float32), wcat)

    ei3 = edge_index.reshape(2, E // B, B)
    degp = _sc_deg(ei3, ones_rows, zeros_deg)
    y1, dis32 = _tc_pre(features, W1, degp)
    disf = dis32.reshape(FR, 128)
    p1 = _sc_prop(y1, ei3, zeros_acc)
    y2f = _tc_mid(p1.reshape(NC, FR, 128), disf)
    p2 = _sc_prop(y2f.reshape(N, H), ei3, zeros_acc)
    zf = _tc_post(p2.reshape(NC, FR, 128), disf, wbd)
    z = zf.reshape(N, H)
    return (z[:, :Z], z[:, Z:])


# speedup vs baseline: 78.3077x; 1.0453x over previous
import functools

import jax
import jax.numpy as jnp
from jax import lax
from jax.experimental import pallas as pl
from jax.experimental.pallas import tpu as pltpu
from jax.experimental.pallas import tpu_sc as plsc

N = 10000
E = 320000
D_IN = 128
H = 32
Z = 16

NC = 2
NS = 16
NW = NC * NS
B = 125
NCHT = E // B // NW
DW = 8
K = 4
A = 2
KD = 8
FR = N * H // 128

_mesh = plsc.VectorSubcoreMesh(core_axis_name="c", subcore_axis_name="s")


@functools.partial(
    pl.kernel,
    out_type=jax.ShapeDtypeStruct((NC, N, DW), jnp.float32),
    mesh=_mesh,
    compiler_params=pltpu.CompilerParams(use_tc_tiling_on_sc=False),
    scratch_types=[
        pltpu.VMEM((B, DW), jnp.float32),
        pltpu.VMEM((NCHT, B), jnp.int32),
        pltpu.VMEM_SHARED((N, DW), jnp.float32),
        pltpu.SemaphoreType.DMA((KD,)),
    ],
)
def _sc_deg(ei_hbm, ones_hbm, zeros_hbm, degp_hbm, onesb, dsti, degsh, ssem):
    c = lax.axis_index("c")
    s = lax.axis_index("s")
    w = c * NS + s
    @pl.when(s < 10)
    def _():
        pltpu.sync_copy(zeros_hbm.at[pl.ds(s * 1000, 1000)],
                        degsh.at[pl.ds(s * 1000, 1000)])
    pltpu.sync_copy(ones_hbm, onesb)
    pltpu.sync_copy(ei_hbm.at[1, pl.ds(w * NCHT, NCHT)], dsti)
    plsc.subcore_barrier()

    @pl.loop(0, NCHT, step=KD)
    def _(j0):
        for b in range(KD):
            j = j0 + b

            @pl.when(j >= KD)
            def _():
                pltpu.make_async_copy(
                    onesb, degsh.at[dsti.at[j - KD]], ssem.at[b]).wait()

            pltpu.async_copy(onesb, degsh.at[dsti.at[j]], ssem.at[b],
                             add=True)

    for b in range(KD):
        pltpu.make_async_copy(
            onesb, degsh.at[dsti.at[NCHT - KD + b]], ssem.at[b]).wait()

    plsc.subcore_barrier()

    @pl.when(s < 10)
    def _():
        pltpu.sync_copy(degsh.at[pl.ds(s * 1000, 1000)],
                        degp_hbm.at[c, pl.ds(s * 1000, 1000)])


@functools.partial(
    pl.kernel,
    out_type=jax.ShapeDtypeStruct((NC, N, H), jnp.float32),
    mesh=_mesh,
    compiler_params=pltpu.CompilerParams(use_tc_tiling_on_sc=False),
    scratch_types=[
        pltpu.VMEM((NCHT, B), jnp.int32),
        pltpu.VMEM((NCHT, B), jnp.int32),
        pltpu.VMEM((K, B, H), jnp.float32),
        pltpu.VMEM_SHARED((N, H), jnp.float32),
        pltpu.SemaphoreType.DMA((K,)),
        pltpu.SemaphoreType.DMA((K,)),
    ],
)
def _sc_prop(y_hbm, ei_hbm, zeros_hbm, p_hbm,
             srci, dsti, rows, accsh, gsem, ssem):
    c = lax.axis_index("c")
    s = lax.axis_index("s")
    w = c * NS + s
    row_base = s * 1000

    @pl.when(jnp.logical_and(s < 10, c == 0))
    def _():
        pltpu.sync_copy(y_hbm.at[pl.ds(row_base, 1000)],
                        accsh.at[pl.ds(row_base, 1000)])

    @pl.when(jnp.logical_and(s < 10, c == 1))
    def _():
        pltpu.sync_copy(zeros_hbm.at[pl.ds(row_base, 1000)],
                        accsh.at[pl.ds(row_base, 1000)])

    pltpu.sync_copy(ei_hbm.at[0, pl.ds(w * NCHT, NCHT)], srci)
    pltpu.sync_copy(ei_hbm.at[1, pl.ds(w * NCHT, NCHT)], dsti)
    plsc.subcore_barrier()

    for j in range(A):
        pltpu.async_copy(y_hbm.at[srci.at[j]], rows.at[j % K], gsem.at[j % K])

    @pl.loop(0, NCHT, step=K)
    def _(j0):
        for b in range(K):
            j = j0 + b
            b2 = (b + A) % K

            @pl.when(j + A >= K)
            def _():
                pltpu.make_async_copy(
                    rows.at[b2], accsh.at[dsti.at[j + A - K]],
                    ssem.at[b2]).wait()

            @pl.when(j + A < NCHT)
            def _():
                pltpu.async_copy(y_hbm.at[srci.at[j + A]], rows.at[b2],
                                 gsem.at[b2])

            pltpu.make_async_copy(
                y_hbm.at[srci.at[j]], rows.at[b], gsem.at[b]).wait()
            pltpu.async_copy(rows.at[b], accsh.at[dsti.at[j]], ssem.at[b],
                             add=True)

    for i in range(K - A):
        j = NCHT - (K - A) + i
        b = j % K
        pltpu.make_async_copy(
            rows.at[b], accsh.at[dsti.at[j]], ssem.at[b]).wait()

    plsc.subcore_barrier()

    @pl.when(s < 10)
    def _():
        pltpu.sync_copy(accsh.at[pl.ds(row_base, 1000)],
                        p_hbm.at[c, pl.ds(row_base, 1000)])


def _tc_pre_body(x_ref, w1_ref, degp_ref, y1_ref, dis_ref):
    deg = degp_ref[0, :, 0:1] + degp_ref[1, :, 0:1] + 1.0
    dis = lax.rsqrt(deg)
    xw = jnp.dot(x_ref[...], w1_ref[...], preferred_element_type=jnp.float32)
    y1_ref[...] = dis * xw
    dis_ref[...] = jnp.broadcast_to(dis, (N, H))


def _tc_mid_body(p_ref, dis_ref, y2_ref):
    dis = dis_ref[...]
    t = p_ref[0:FR] + p_ref[FR:2 * FR]
    y2_ref[...] = dis * jnp.maximum(dis * t, 0.0)


def _tc_post_body(p_ref, dis_ref, wbd_ref, zf_ref):
    sfin = dis_ref[...] * (p_ref[0:FR] + p_ref[FR:2 * FR])
    zf_ref[...] = jnp.dot(sfin, wbd_ref[...],
                          preferred_element_type=jnp.float32)


_tc_pre = pl.pallas_call(
    _tc_pre_body,
    out_shape=(
        jax.ShapeDtypeStruct((N, H), jnp.float32),
        jax.ShapeDtypeStruct((N, H), jnp.float32),
    ),
)

_tc_mid = pl.pallas_call(
    _tc_mid_body,
    out_shape=jax.ShapeDtypeStruct((FR, 128), jnp.float32),
)

_tc_post = pl.pallas_call(
    _tc_post_body,
    out_shape=jax.ShapeDtypeStruct((FR, 128), jnp.float32),
)


@jax.jit
def kernel(features, edge_index, W1, W_mean, W_var):
    ones_rows = jnp.ones((B, DW), jnp.float32)
    zeros_deg = jnp.zeros((N, DW), jnp.float32)
    zeros_acc = jnp.zeros((N, H), jnp.float32)
    wcat = jnp.concatenate([W_mean, W_var], axis=1)
    wbd = jnp.kron(jnp.eye(4, dtype=jnp.float32), wcat)

    ei3 = edge_index.reshape(2, E // B, B)
    degp = _sc_deg(ei3, ones_rows, zeros_deg)
    y1, dis32 = _tc_pre(features, W1, degp)
    disf = dis32.reshape(FR, 128)
    p1 = _sc_prop(y1, ei3, zeros_acc)
    y2f = _tc_mid(p1.reshape(NC * FR, 128), disf)
    p2 = _sc_prop(y2f.reshape(N, H), ei3, zeros_acc)
    zf = _tc_post(p2.reshape(NC * FR, 128), disf, wbd)
    z = zf.reshape(N, H)
    return (z[:, :Z], z[:, Z:])
